# baseline (speedup 1.0000x reference)
import math
import jax
import jax.numpy as jnp
from jax import lax
from jax.experimental import pallas as pl
from jax.experimental.pallas import tpu as pltpu
from jax.experimental.pallas import tpu_sc as plsc

N_NODES = 10000
EPS = 1e-7
RATIO = 0.8

_NC, _NS = 2, 16
_E = 320000

_D2 = 128
_CPT = 4
_NP2 = 8016
_ECH = 8000
_NCHUNK = _E // _ECH
_GRP = _ECH // 4

_D1 = 4
_NP1 = 10016
_ES = 8
_EPT = _E // _ES
_ECH1 = 8000
_NCH1 = _EPT // _ECH1
_GRP1 = _ECH1 // 16

_K1 = 8000
_PPT = _K1 // 32
_EPT_B = _E // 32
_ECHB = 2000
_NCHB = _EPT_B // _ECHB
_GRPB = _ECHB // 16

_UNROLL = 8


def _fast_exp(x):
    y = jnp.maximum(x * 1.4426950408889634, -125.0)
    t = y + 12582912.0
    ni = plsc.bitcast(t, jnp.int32) - 0x4B400000
    f = y - (t - 12582912.0)
    p = 0.0013333558146428443 * f + 0.009618129107628477
    p = p * f + 0.05550410866482158
    p = p * f + 0.24022650695910072
    p = p * f + 0.6931471805599453
    p = p * f + 1.0
    scale = plsc.bitcast(lax.shift_left(ni + 127, 23), jnp.float32)
    return p * scale


def _conv2_body(xt_ref, src_ref, dst_ref, c_ref, den_ref, num_ref, xv, srcv, dstv, cv, denv, numv):
    wid = lax.axis_index("s") * _NC + lax.axis_index("c")
    ch0 = wid * _CPT

    pltpu.sync_copy(xt_ref.at[pl.ds(pl.multiple_of(ch0 * _NP2, 8), _CPT * _NP2)], xv)
    pltpu.sync_copy(c_ref, cv)
    cls = [plsc.load_gather(cv, [jnp.full((16,), 0, jnp.int32) + (ch0 + q)])
           for q in range(_CPT)]

    def zbody(i, _):
        o = pl.ds(pl.multiple_of(i * 16, 8), 16)
        denv[o] = jnp.zeros((16,), jnp.float32)
        numv[o] = jnp.zeros((16,), jnp.float32)
        return 0
    lax.fori_loop(0, _CPT * _NP2 // 16, zbody, 0, unroll=_UNROLL)

    def chunk(ic, _):
        off = pl.ds(pl.multiple_of(ic * _ECH, 8), _ECH)
        pltpu.sync_copy(src_ref.at[off], srcv)
        pltpu.sync_copy(dst_ref.at[off], dstv)

        @plsc.parallel_loop(0, _ECH // 16, 1, unroll=4, carry=jnp.int32(0))
        def grp(g, j):
            sl = pl.ds(pl.multiple_of(g * 16, 8), 16)
            sv = srcv[sl]
            dv = dstv[sl]
            for q in range(_CPT):
                xg = plsc.load_gather(xv, [sv + q * _NP2])
                msg = jnp.maximum(xg, 0.0) + EPS
                w = _fast_exp(msg - cls[q])
                sidx = dv + q * _NP2
                plsc.addupdate_scatter(denv, [sidx], w)
                plsc.addupdate_scatter(numv, [sidx], w * msg)
            return j
        return 0
    lax.fori_loop(0, _NCHUNK, chunk, 0)

    osl = pl.ds(pl.multiple_of(ch0 * _NP2, 8), _CPT * _NP2)
    pltpu.sync_copy(denv, den_ref.at[osl])
    pltpu.sync_copy(numv, num_ref.at[osl])


def _conv2_aggr_sc(xt_flat, src, dst, c):
    mesh = plsc.VectorSubcoreMesh(core_axis_name="c", subcore_axis_name="s",
                                  num_cores=_NC, num_subcores=_NS)
    f = pl.kernel(
        _conv2_body,
        out_type=(jax.ShapeDtypeStruct((_D2 * _NP2,), jnp.float32),
                  jax.ShapeDtypeStruct((_D2 * _NP2,), jnp.float32)),
        mesh=mesh,
        compiler_params=pltpu.CompilerParams(needs_layout_passes=False),
        scratch_types=[
            pltpu.VMEM((_CPT * _NP2,), jnp.float32),
            pltpu.VMEM((_ECH,), jnp.int32),
            pltpu.VMEM((_ECH,), jnp.int32),
            pltpu.VMEM((_D2,), jnp.float32),
            pltpu.VMEM((_CPT * _NP2,), jnp.float32),
            pltpu.VMEM((_CPT * _NP2,), jnp.float32),
        ],
    )
    return f(xt_flat, src, dst, c)


def _conv1_body(xt_ref, ea_ref, src_ref, dst_ref, cb_ref, out_ref,
                xv, srcv, dstv, eav, cv, denv, numv):
    wid = lax.axis_index("s") * _NC + lax.axis_index("c")
    q = wid & 3
    es = lax.shift_right_logical(wid, 2)

    pltpu.sync_copy(xt_ref.at[pl.ds(pl.multiple_of(q * _NP1, 8), _NP1)], xv)
    pltpu.sync_copy(cb_ref.at[wid], cv)
    cl = cv[...]

    def zbody(i, _):
        o = pl.ds(pl.multiple_of(i * 16, 8), 16)
        denv[o] = jnp.zeros((16,), jnp.float32)
        numv[o] = jnp.zeros((16,), jnp.float32)
        return 0
    lax.fori_loop(0, _NP1 // 16, zbody, 0, unroll=_UNROLL)

    ebase = es * _EPT

    def chunk(ic, _):
        off = ebase + ic * _ECH1
        pltpu.sync_copy(src_ref.at[pl.ds(pl.multiple_of(off, 8), _ECH1)], srcv)
        pltpu.sync_copy(dst_ref.at[pl.ds(pl.multiple_of(off, 8), _ECH1)], dstv)
        pltpu.sync_copy(ea_ref.at[pl.ds(pl.multiple_of(q * _E + off, 8), _ECH1)], eav)

        @plsc.parallel_loop(0, _GRP1, 1, unroll=8, carry=jnp.int32(0))
        def grp(g, j):
            sl = pl.ds(pl.multiple_of(g * 16, 8), 16)
            sv = srcv[sl]
            dv = dstv[sl]
            ev = eav[sl]
            xg = plsc.load_gather(xv, [sv])
            msg = jnp.maximum(xg + ev, 0.0) + EPS
            w = _fast_exp(msg - cl)
            plsc.addupdate_scatter(denv, [dv], w)
            plsc.addupdate_scatter(numv, [dv], w * msg)
            return j
        return 0
    lax.fori_loop(0, _NCH1, chunk, 0)

    obase = pl.multiple_of(wid * 2 * _NP1, 8)
    pltpu.sync_copy(denv, out_ref.at[pl.ds(obase, _NP1)])
    pltpu.sync_copy(numv, out_ref.at[pl.ds(pl.multiple_of(obase + _NP1, 8), _NP1)])


def _conv1_aggr_sc(xt_flat, ea_flat, src, dst, cb):
    mesh = plsc.VectorSubcoreMesh(core_axis_name="c", subcore_axis_name="s",
                                  num_cores=_NC, num_subcores=_NS)
    f = pl.kernel(
        _conv1_body,
        out_type=jax.ShapeDtypeStruct((32 * 2 * _NP1,), jnp.float32),
        mesh=mesh,
        compiler_params=pltpu.CompilerParams(needs_layout_passes=False),
        scratch_types=[
            pltpu.VMEM((_NP1,), jnp.float32),
            pltpu.VMEM((_ECH1,), jnp.int32),
            pltpu.VMEM((_ECH1,), jnp.int32),
            pltpu.VMEM((_ECH1,), jnp.float32),
            pltpu.VMEM((16,), jnp.float32),
            pltpu.VMEM((_NP1,), jnp.float32),
            pltpu.VMEM((_NP1,), jnp.float32),
        ],
    )
    return f(xt_flat, ea_flat, src, dst, cb)


def _pool1_body(ht_ref, sc_ref, perm_ref, src_ref, dst_ref,
                xnt_ref, s2_ref, d2_ref, st_ref,
                xv, scv, permv, nidxv, onv, srcv, dstv, s2v, d2v, statv):
    wid = lax.axis_index("s") * _NC + lax.axis_index("c")
    ch0 = wid * _CPT
    iota = lax.iota(jnp.int32, 16)

    pltpu.sync_copy(ht_ref.at[pl.ds(pl.multiple_of(ch0 * _NP1, 8), _CPT * _NP1)], xv)
    pltpu.sync_copy(sc_ref, scv)
    pltpu.sync_copy(perm_ref, permv)

    def zb(i, _):
        nidxv[pl.ds(pl.multiple_of(i * 16, 8), 16)] = jnp.full((16,), _K1, jnp.int32)
        return 0
    lax.fori_loop(0, _NP1 // 16, zb, 0, unroll=_UNROLL)

    def sb(g, _):
        pv = permv[pl.ds(pl.multiple_of(g * 16, 8), 16)]
        plsc.store_scatter(nidxv, [pv], g * 16 + iota)
        return 0
    lax.fori_loop(0, _K1 // 16, sb, 0, unroll=_UNROLL)

    for q in range(_CPT):
        def gb(g, carry):
            mx, sm = carry
            pv = permv[pl.ds(pl.multiple_of(g * 16, 8), 16)]
            s = plsc.load_gather(scv, [pv])
            xg = plsc.load_gather(xv, [q * _NP1 + pv])
            v = xg * s
            onv[pl.ds(pl.multiple_of(q * _NP2 + g * 16, 8), 16)] = v
            return (jnp.maximum(mx, v), sm + v)
        mx, sm = lax.fori_loop(0, _K1 // 16, gb,
                               (jnp.full((16,), -3.4e38, jnp.float32),
                                jnp.zeros((16,), jnp.float32)), unroll=_UNROLL)
        onv[pl.ds(q * _NP2 + _K1, 16)] = jnp.zeros((16,), jnp.float32)
        statv[pl.ds(q * 32, 16)] = mx
        statv[pl.ds(q * 32 + 16, 16)] = sm

    pltpu.sync_copy(onv, xnt_ref.at[pl.ds(pl.multiple_of(ch0 * _NP2, 8), _CPT * _NP2)])
    pltpu.sync_copy(statv, st_ref.at[wid])

    ebase = wid * _EPT_B

    def chunk(ic, _):
        off = ebase + ic * _ECHB
        osl = pl.ds(pl.multiple_of(off, 8), _ECHB)
        pltpu.sync_copy(src_ref.at[osl], srcv)
        pltpu.sync_copy(dst_ref.at[osl], dstv)

        def grp(g, _):
            sl = pl.ds(pl.multiple_of(g * 16, 8), 16)
            ns = plsc.load_gather(nidxv, [srcv[sl]])
            nd = plsc.load_gather(nidxv, [dstv[sl]])
            bad = (ns == _K1) | (nd == _K1)
            s2v[sl] = jnp.where(bad, _K1, ns)
            d2v[sl] = jnp.where(bad, _K1, nd)
            return 0
        lax.fori_loop(0, _GRPB, grp, 0, unroll=_UNROLL)
        pltpu.sync_copy(s2v, s2_ref.at[osl])
        pltpu.sync_copy(d2v, d2_ref.at[osl])
        return 0
    lax.fori_loop(0, _NCHB, chunk, 0)


def _pool1_sc(ht_flat, score, perm, src, dst):
    mesh = plsc.VectorSubcoreMesh(core_axis_name="c", subcore_axis_name="s",
                                  num_cores=_NC, num_subcores=_NS)
    f = pl.kernel(
        _pool1_body,
        out_type=(jax.ShapeDtypeStruct((_D2 * _NP2,), jnp.float32),
                  jax.ShapeDtypeStruct((_E,), jnp.int32),
                  jax.ShapeDtypeStruct((_E,), jnp.int32),
                  jax.ShapeDtypeStruct((32, 128), jnp.float32)),
        mesh=mesh,
        compiler_params=pltpu.CompilerParams(needs_layout_passes=False),
        scratch_types=[
            pltpu.VMEM((_CPT * _NP1,), jnp.float32),
            pltpu.VMEM((_NP1,), jnp.float32),
            pltpu.VMEM((_K1,), jnp.int32),
            pltpu.VMEM((_NP1,), jnp.int32),
            pltpu.VMEM((_CPT * _NP2,), jnp.float32),
            pltpu.VMEM((_ECHB,), jnp.int32),
            pltpu.VMEM((_ECHB,), jnp.int32),
            pltpu.VMEM((_ECHB,), jnp.int32),
            pltpu.VMEM((_ECHB,), jnp.int32),
            pltpu.VMEM((128,), jnp.float32),
        ],
    )
    return f(ht_flat, score, perm, src, dst)


def _k2_body(parts_ref, xt_ref, w1_ref, b1_ref, g_ref, bb_ref, w2_ref, b2_ref, pn_ref,
             ht_ref, sc_ref):
    ps = parts_ref[...]
    dn = jnp.sum(ps.reshape(_ES, _D1, 2, _NP1), axis=0)
    aggr = dn[:, 1, :] / jnp.maximum(dn[:, 0, :], 1e-30)
    out1 = xt_ref[...] + aggr
    h = jnp.dot(w1_ref[...], out1, preferred_element_type=jnp.float32) + b1_ref[...]
    mask = lax.broadcasted_iota(jnp.int32, (1, _NP1), 1) < N_NODES
    hm = jnp.where(mask, h, 0.0)
    mu = jnp.sum(hm, axis=1, keepdims=True) / N_NODES
    d = jnp.where(mask, h - mu, 0.0)
    var = jnp.sum(d * d, axis=1, keepdims=True) / N_NODES
    hn = (h - mu) * lax.rsqrt(var + 1e-5) * g_ref[...] + bb_ref[...]
    hr = jnp.maximum(hn, 0.0)
    h2 = jnp.maximum(jnp.dot(w2_ref[...], hr, preferred_element_type=jnp.float32) + b2_ref[...], 0.0)
    ht_ref[...] = h2
    s = jnp.dot(pn_ref[...], h2, preferred_element_type=jnp.float32)
    sc_ref[...] = jnp.where(mask, jnp.tanh(s), -1e30)


def _k4_body(xnt_ref, den_ref, num_ref, w1_ref, b1_ref, g_ref, bb_ref, w2_ref, b2_ref, pn_ref,
             ht_ref, sc_ref):
    aggr = num_ref[...] / jnp.maximum(den_ref[...], 1e-30)
    out2 = xnt_ref[...] + aggr
    h = jnp.dot(w1_ref[...], out2, preferred_element_type=jnp.float32) + b1_ref[...]
    mask = lax.broadcasted_iota(jnp.int32, (1, _NP2), 1) < _K1
    hm = jnp.where(mask, h, 0.0)
    mu = jnp.sum(hm, axis=1, keepdims=True) / _K1
    d = jnp.where(mask, h - mu, 0.0)
    var = jnp.sum(d * d, axis=1, keepdims=True) / _K1
    hn = (h - mu) * lax.rsqrt(var + 1e-5) * g_ref[...] + bb_ref[...]
    hr = jnp.maximum(hn, 0.0)
    h2 = jnp.maximum(jnp.dot(w2_ref[...], hr, preferred_element_type=jnp.float32) + b2_ref[...], 0.0)
    ht_ref[...] = h2
    s = jnp.dot(pn_ref[...], h2, preferred_element_type=jnp.float32)
    sc_ref[...] = jnp.where(mask, jnp.tanh(s), -1e30)


def _k6_body(ht_ref, sc_ref, mask_ref, x1_ref, l1w_ref, l1b_ref, l2w_ref, l2b_ref, out_ref):
    v = ht_ref[...] * sc_ref[...]
    m = mask_ref[...]
    x2max = jnp.max(jnp.where(m > 0, v, -3.4e38), axis=1, keepdims=True)
    x2mean = jnp.sum(v * m, axis=1, keepdims=True) / _K2
    z = x1_ref[...] + jnp.concatenate([x2max, x2mean], axis=0)
    h = jnp.maximum(jnp.dot(l1w_ref[...], z, preferred_element_type=jnp.float32) + l1b_ref[...], 0.0)
    out_ref[...] = jnp.dot(l2w_ref[...], h, preferred_element_type=jnp.float32) + l2b_ref[...]


_K2 = 6400


def kernel(x, edge_index, edge_attr, batch, c1_w1, c1_b1, c1_g, c1_bb, c1_w2, c1_b2, p1_w, c2_w1, c2_b1, c2_g, c2_bb, c2_w2, c2_b2, p2_w, l1_w, l1_b, l2_w, l2_b):
    src, dst = edge_index[0], edge_index[1]
    f32 = jnp.float32

    xt1 = jnp.zeros((_D1, _NP1), f32).at[:3, :N_NODES].set(x.T)
    ea_t = edge_attr.T.reshape(-1)
    c1 = jnp.maximum(jnp.max(xt1, axis=1) + jnp.max(edge_attr, axis=0), 0.0) + EPS
    cb32 = jnp.tile(jnp.broadcast_to(c1[:, None], (_D1, 16)), (8, 1))

    parts = _conv1_aggr_sc(xt1.reshape(-1), ea_t, src, dst, cb32).reshape(32, 2, _NP1)

    p1n = (p1_w / jnp.linalg.norm(p1_w))[None, :]
    h1t, sc1 = pl.pallas_call(
        _k2_body,
        out_shape=(jax.ShapeDtypeStruct((_D2, _NP1), f32),
                   jax.ShapeDtypeStruct((1, _NP1), f32)),
    )(parts, xt1, c1_w1, c1_b1[:, None], c1_g[:, None], c1_bb[:, None],
      c1_w2, c1_b2[:, None], p1n)

    perm1 = jax.lax.top_k(sc1[0], _K1)[1]

    xnt, src2, dst2, st1 = _pool1_sc(h1t.reshape(-1), sc1[0], perm1, src, dst)
    st1 = st1.reshape(32, _CPT, 2, 16)
    x1max = jnp.max(st1[:, :, 0, :], axis=-1).reshape(_D2)
    x1mean = (jnp.sum(st1[:, :, 1, :], axis=-1) / _K1).reshape(_D2)
    x1c = jnp.concatenate([x1max, x1mean])[:, None]
    c2 = jnp.maximum(x1max, 0.0) + EPS

    den2, num2 = _conv2_aggr_sc(xnt, src2, dst2, c2)

    p2n = (p2_w / jnp.linalg.norm(p2_w))[None, :]
    h2t, sc2 = pl.pallas_call(
        _k4_body,
        out_shape=(jax.ShapeDtypeStruct((_D2, _NP2), f32),
                   jax.ShapeDtypeStruct((1, _NP2), f32)),
    )(xnt.reshape(_D2, _NP2), den2.reshape(_D2, _NP2), num2.reshape(_D2, _NP2),
      c2_w1, c2_b1[:, None], c2_g[:, None], c2_bb[:, None],
      c2_w2, c2_b2[:, None], p2n)

    perm2 = jax.lax.top_k(sc2[0], _K2)[1]
    mask2 = jnp.zeros((1, _NP2), f32).at[0, perm2].set(1.0)

    out = pl.pallas_call(
        _k6_body,
        out_shape=jax.ShapeDtypeStruct((2, 1), f32),
    )(h2t, sc2, mask2, x1c, l1_w, l1_b[:, None], l2_w, l2_b[:, None])
    return out.reshape(1, 2)


# speedup vs baseline: 1.0643x; 1.0643x over previous
import math
import jax
import jax.numpy as jnp
from jax import lax
from jax.experimental import pallas as pl
from jax.experimental.pallas import tpu as pltpu
from jax.experimental.pallas import tpu_sc as plsc

N_NODES = 10000
EPS = 1e-7
RATIO = 0.8

_NC, _NS = 2, 16
_E = 320000

_D2 = 128
_CPT = 4
_NP2 = 8016
_ECH = 8000
_NCHUNK = _E // _ECH
_GRP = _ECH // 4

_D1 = 4
_NP1 = 10016
_ES = 8
_EPT = _E // _ES
_ECH1 = 8000
_NCH1 = _EPT // _ECH1
_GRP1 = _ECH1 // 16

_K1 = 8000
_PPT = _K1 // 32
_EPT_B = _E // 32
_ECHB = 2000
_NCHB = _EPT_B // _ECHB
_GRPB = _ECHB // 16

_UNROLL = 8


def _fast_exp(x):
    y = jnp.maximum(x * 1.4426950408889634, -125.0)
    t = y + 12582912.0
    ni = plsc.bitcast(t, jnp.int32) - 0x4B400000
    f = y - (t - 12582912.0)
    p = 0.0013333558146428443 * f + 0.009618129107628477
    p = p * f + 0.05550410866482158
    p = p * f + 0.24022650695910072
    p = p * f + 0.6931471805599453
    p = p * f + 1.0
    scale = plsc.bitcast(lax.shift_left(ni + 127, 23), jnp.float32)
    return p * scale


def _conv2_body(xt_ref, src_ref, dst_ref, c_ref, out_ref, xv, srcv, dstv, cv, denv, numv):
    wid = lax.axis_index("s") * _NC + lax.axis_index("c")
    ch0 = wid * _CPT

    pltpu.sync_copy(xt_ref.at[pl.ds(pl.multiple_of(ch0 * _NP2, 8), _CPT * _NP2)], xv)
    pltpu.sync_copy(c_ref, cv)
    cls = [plsc.load_gather(cv, [jnp.full((16,), 0, jnp.int32) + (ch0 + q)])
           for q in range(_CPT)]

    def zbody(i, _):
        o = pl.ds(pl.multiple_of(i * 16, 8), 16)
        denv[o] = jnp.zeros((16,), jnp.float32)
        numv[o] = jnp.zeros((16,), jnp.float32)
        return 0
    lax.fori_loop(0, _CPT * _NP2 // 16, zbody, 0, unroll=_UNROLL)

    def chunk(ic, _):
        off = pl.ds(pl.multiple_of(ic * _ECH, 8), _ECH)
        pltpu.sync_copy(src_ref.at[off], srcv)
        pltpu.sync_copy(dst_ref.at[off], dstv)

        @plsc.parallel_loop(0, _ECH // 16, 1, unroll=4, carry=jnp.int32(0))
        def grp(g, j):
            sl = pl.ds(pl.multiple_of(g * 16, 8), 16)
            sv = srcv[sl]
            dv = dstv[sl]
            for q in range(_CPT):
                xg = plsc.load_gather(xv, [sv + q * _NP2])
                msg = jnp.maximum(xg, 0.0) + EPS
                w = jnp.exp(msg - cls[q])
                sidx = dv + q * _NP2
                plsc.addupdate_scatter(denv, [sidx], w)
                plsc.addupdate_scatter(numv, [sidx], w * msg)
            return j
        return 0
    lax.fori_loop(0, _NCHUNK, chunk, 0)

    def dbody(i, _):
        o = pl.ds(pl.multiple_of(i * 16, 8), 16)
        numv[o] = numv[o] / jnp.maximum(denv[o], 1e-30)
        return 0
    lax.fori_loop(0, _CPT * _NP2 // 16, dbody, 0, unroll=_UNROLL)
    pltpu.sync_copy(numv, out_ref.at[pl.ds(pl.multiple_of(ch0 * _NP2, 8), _CPT * _NP2)])


def _conv2_aggr_sc(xt_flat, src, dst, c):
    mesh = plsc.VectorSubcoreMesh(core_axis_name="c", subcore_axis_name="s",
                                  num_cores=_NC, num_subcores=_NS)
    f = pl.kernel(
        _conv2_body,
        out_type=jax.ShapeDtypeStruct((_D2 * _NP2,), jnp.float32),
        mesh=mesh,
        compiler_params=pltpu.CompilerParams(needs_layout_passes=False),
        scratch_types=[
            pltpu.VMEM((_CPT * _NP2,), jnp.float32),
            pltpu.VMEM((_ECH,), jnp.int32),
            pltpu.VMEM((_ECH,), jnp.int32),
            pltpu.VMEM((_D2,), jnp.float32),
            pltpu.VMEM((_CPT * _NP2,), jnp.float32),
            pltpu.VMEM((_CPT * _NP2,), jnp.float32),
        ],
    )
    return f(xt_flat, src, dst, c)


def _conv1_body(xt_ref, ea_ref, src_ref, dst_ref, cb_ref, out_ref,
                xv, srcv, dstv, eav, cv, denv, numv):
    wid = lax.axis_index("s") * _NC + lax.axis_index("c")
    q = wid & 3
    es = lax.shift_right_logical(wid, 2)

    pltpu.sync_copy(xt_ref.at[pl.ds(pl.multiple_of(q * _NP1, 8), _NP1)], xv)
    pltpu.sync_copy(cb_ref.at[wid], cv)
    cl = cv[...]

    def zbody(i, _):
        o = pl.ds(pl.multiple_of(i * 16, 8), 16)
        denv[o] = jnp.zeros((16,), jnp.float32)
        numv[o] = jnp.zeros((16,), jnp.float32)
        return 0
    lax.fori_loop(0, _NP1 // 16, zbody, 0, unroll=_UNROLL)

    ebase = es * _EPT

    def chunk(ic, _):
        off = ebase + ic * _ECH1
        pltpu.sync_copy(src_ref.at[pl.ds(pl.multiple_of(off, 8), _ECH1)], srcv)
        pltpu.sync_copy(dst_ref.at[pl.ds(pl.multiple_of(off, 8), _ECH1)], dstv)
        pltpu.sync_copy(ea_ref.at[pl.ds(pl.multiple_of(q * _E + off, 8), _ECH1)], eav)

        @plsc.parallel_loop(0, _GRP1, 1, unroll=8, carry=jnp.int32(0))
        def grp(g, j):
            sl = pl.ds(pl.multiple_of(g * 16, 8), 16)
            sv = srcv[sl]
            dv = dstv[sl]
            ev = eav[sl]
            xg = plsc.load_gather(xv, [sv])
            msg = jnp.maximum(xg + ev, 0.0) + EPS
            w = _fast_exp(msg - cl)
            plsc.addupdate_scatter(denv, [dv], w)
            plsc.addupdate_scatter(numv, [dv], w * msg)
            return j
        return 0
    lax.fori_loop(0, _NCH1, chunk, 0)

    obase = pl.multiple_of(wid * 2 * _NP1, 8)
    pltpu.sync_copy(denv, out_ref.at[pl.ds(obase, _NP1)])
    pltpu.sync_copy(numv, out_ref.at[pl.ds(pl.multiple_of(obase + _NP1, 8), _NP1)])


def _conv1_aggr_sc(xt_flat, ea_flat, src, dst, cb):
    mesh = plsc.VectorSubcoreMesh(core_axis_name="c", subcore_axis_name="s",
                                  num_cores=_NC, num_subcores=_NS)
    f = pl.kernel(
        _conv1_body,
        out_type=jax.ShapeDtypeStruct((32 * 2 * _NP1,), jnp.float32),
        mesh=mesh,
        compiler_params=pltpu.CompilerParams(needs_layout_passes=False),
        scratch_types=[
            pltpu.VMEM((_NP1,), jnp.float32),
            pltpu.VMEM((_ECH1,), jnp.int32),
            pltpu.VMEM((_ECH1,), jnp.int32),
            pltpu.VMEM((_ECH1,), jnp.float32),
            pltpu.VMEM((16,), jnp.float32),
            pltpu.VMEM((_NP1,), jnp.float32),
            pltpu.VMEM((_NP1,), jnp.float32),
        ],
    )
    return f(xt_flat, ea_flat, src, dst, cb)


def _pool1_body(ht_ref, sc_ref, perm_ref, src_ref, dst_ref,
                xnt_ref, s2_ref, d2_ref, st_ref,
                xv, scv, permv, nidxv, onv, srcv, dstv, s2v, d2v, statv):
    wid = lax.axis_index("s") * _NC + lax.axis_index("c")
    ch0 = wid * _CPT
    iota = lax.iota(jnp.int32, 16)

    pltpu.sync_copy(ht_ref.at[pl.ds(pl.multiple_of(ch0 * _NP1, 8), _CPT * _NP1)], xv)
    pltpu.sync_copy(sc_ref, scv)
    pltpu.sync_copy(perm_ref, permv)

    def zb(i, _):
        nidxv[pl.ds(pl.multiple_of(i * 16, 8), 16)] = jnp.full((16,), _K1, jnp.int32)
        return 0
    lax.fori_loop(0, _NP1 // 16, zb, 0, unroll=_UNROLL)

    def sb(g, _):
        pv = permv[pl.ds(pl.multiple_of(g * 16, 8), 16)]
        plsc.store_scatter(nidxv, [pv], g * 16 + iota)
        return 0
    lax.fori_loop(0, _K1 // 16, sb, 0, unroll=_UNROLL)

    for q in range(_CPT):
        def gb(g, carry):
            mx, sm = carry
            pv = permv[pl.ds(pl.multiple_of(g * 16, 8), 16)]
            s = plsc.load_gather(scv, [pv])
            xg = plsc.load_gather(xv, [q * _NP1 + pv])
            v = xg * s
            onv[pl.ds(pl.multiple_of(q * _NP2 + g * 16, 8), 16)] = v
            return (jnp.maximum(mx, v), sm + v)
        mx, sm = lax.fori_loop(0, _K1 // 16, gb,
                               (jnp.full((16,), -3.4e38, jnp.float32),
                                jnp.zeros((16,), jnp.float32)), unroll=_UNROLL)
        onv[pl.ds(q * _NP2 + _K1, 16)] = jnp.zeros((16,), jnp.float32)
        statv[pl.ds(q * 32, 16)] = mx
        statv[pl.ds(q * 32 + 16, 16)] = sm

    pltpu.sync_copy(onv, xnt_ref.at[pl.ds(pl.multiple_of(ch0 * _NP2, 8), _CPT * _NP2)])
    pltpu.sync_copy(statv, st_ref.at[wid])

    ebase = wid * _EPT_B

    def chunk(ic, _):
        off = ebase + ic * _ECHB
        osl = pl.ds(pl.multiple_of(off, 8), _ECHB)
        pltpu.sync_copy(src_ref.at[osl], srcv)
        pltpu.sync_copy(dst_ref.at[osl], dstv)

        def grp(g, _):
            sl = pl.ds(pl.multiple_of(g * 16, 8), 16)
            ns = plsc.load_gather(nidxv, [srcv[sl]])
            nd = plsc.load_gather(nidxv, [dstv[sl]])
            bad = (ns == _K1) | (nd == _K1)
            s2v[sl] = jnp.where(bad, _K1, ns)
            d2v[sl] = jnp.where(bad, _K1, nd)
            return 0
        lax.fori_loop(0, _GRPB, grp, 0, unroll=_UNROLL)
        pltpu.sync_copy(s2v, s2_ref.at[osl])
        pltpu.sync_copy(d2v, d2_ref.at[osl])
        return 0
    lax.fori_loop(0, _NCHB, chunk, 0)


def _pool1_sc(ht_flat, score, perm, src, dst):
    mesh = plsc.VectorSubcoreMesh(core_axis_name="c", subcore_axis_name="s",
                                  num_cores=_NC, num_subcores=_NS)
    f = pl.kernel(
        _pool1_body,
        out_type=(jax.ShapeDtypeStruct((_D2 * _NP2,), jnp.float32),
                  jax.ShapeDtypeStruct((_E,), jnp.int32),
                  jax.ShapeDtypeStruct((_E,), jnp.int32),
                  jax.ShapeDtypeStruct((32, 128), jnp.float32)),
        mesh=mesh,
        compiler_params=pltpu.CompilerParams(needs_layout_passes=False),
        scratch_types=[
            pltpu.VMEM((_CPT * _NP1,), jnp.float32),
            pltpu.VMEM((_NP1,), jnp.float32),
            pltpu.VMEM((_K1,), jnp.int32),
            pltpu.VMEM((_NP1,), jnp.int32),
            pltpu.VMEM((_CPT * _NP2,), jnp.float32),
            pltpu.VMEM((_ECHB,), jnp.int32),
            pltpu.VMEM((_ECHB,), jnp.int32),
            pltpu.VMEM((_ECHB,), jnp.int32),
            pltpu.VMEM((_ECHB,), jnp.int32),
            pltpu.VMEM((128,), jnp.float32),
        ],
    )
    return f(ht_flat, score, perm, src, dst)


def _k2_body(parts_ref, xt_ref, w1_ref, b1_ref, g_ref, bb_ref, w2_ref, b2_ref, pn_ref, nrm_ref,
             ht_ref, sc_ref):
    ps = parts_ref[...]
    dn = jnp.sum(ps.reshape(_ES, _D1, 2, _NP1), axis=0)
    aggr = dn[:, 1, :] / jnp.maximum(dn[:, 0, :], 1e-30)
    out1 = xt_ref[...] + aggr
    h = jnp.dot(w1_ref[...].astype(jnp.bfloat16), out1.astype(jnp.bfloat16),
                preferred_element_type=jnp.float32) + b1_ref[...]
    mask = lax.broadcasted_iota(jnp.int32, (1, _NP1), 1) < N_NODES
    hm = jnp.where(mask, h, 0.0)
    mu = jnp.sum(hm, axis=1, keepdims=True) / N_NODES
    d = jnp.where(mask, h - mu, 0.0)
    var = jnp.sum(d * d, axis=1, keepdims=True) / N_NODES
    v = var + 1e-5
    r = lax.rsqrt(v)
    r = r * (1.5 - 0.5 * v * r * r)
    hn = (h - mu) * r * g_ref[...] + bb_ref[...]
    hr = jnp.maximum(hn, 0.0)
    h2 = jnp.maximum(jnp.dot(w2_ref[...].astype(jnp.bfloat16), hr.astype(jnp.bfloat16),
                             preferred_element_type=jnp.float32) + b2_ref[...], 0.0)
    ht_ref[...] = h2
    s = jnp.dot(pn_ref[...].astype(jnp.bfloat16), h2.astype(jnp.bfloat16),
                preferred_element_type=jnp.float32) / nrm_ref[...]
    sc_ref[...] = jnp.where(mask, jnp.tanh(s), -1e30)


def _k4_body(xnt_ref, aggr_ref, w1_ref, b1_ref, g_ref, bb_ref, w2_ref, b2_ref, pn_ref, nrm_ref,
             ht_ref, sc_ref):
    out2 = xnt_ref[...] + aggr_ref[...]
    h = jnp.dot(w1_ref[...].astype(jnp.bfloat16), out2.astype(jnp.bfloat16),
                preferred_element_type=jnp.float32) + b1_ref[...]
    mask = lax.broadcasted_iota(jnp.int32, (1, _NP2), 1) < _K1
    hm = jnp.where(mask, h, 0.0)
    mu = jnp.sum(hm, axis=1, keepdims=True) / _K1
    d = jnp.where(mask, h - mu, 0.0)
    var = jnp.sum(d * d, axis=1, keepdims=True) / _K1
    v = var + 1e-5
    r = lax.rsqrt(v)
    r = r * (1.5 - 0.5 * v * r * r)
    hn = (h - mu) * r * g_ref[...] + bb_ref[...]
    hr = jnp.maximum(hn, 0.0)
    h2 = jnp.maximum(jnp.dot(w2_ref[...].astype(jnp.bfloat16), hr.astype(jnp.bfloat16),
                             preferred_element_type=jnp.float32) + b2_ref[...], 0.0)
    ht_ref[...] = h2
    s = jnp.dot(pn_ref[...].astype(jnp.bfloat16), h2.astype(jnp.bfloat16),
                preferred_element_type=jnp.float32) / nrm_ref[...]
    sc_ref[...] = jnp.where(mask, jnp.tanh(s), -1e30)


def _k6_body(ht_ref, sc_ref, mask_ref, x1_ref, l1w_ref, l1b_ref, l2w_ref, l2b_ref, out_ref):
    v = ht_ref[...] * sc_ref[...]
    m = mask_ref[...]
    x2max = jnp.max(jnp.where(m > 0, v, -3.4e38), axis=1, keepdims=True)
    x2mean = jnp.sum(v * m, axis=1, keepdims=True) / _K2
    z = x1_ref[...] + jnp.concatenate([x2max, x2mean], axis=0)
    h = jnp.maximum(jnp.dot(l1w_ref[...].astype(jnp.bfloat16), z.astype(jnp.bfloat16),
                            preferred_element_type=jnp.float32) + l1b_ref[...], 0.0)
    out_ref[...] = jnp.dot(l2w_ref[...].astype(jnp.bfloat16), h.astype(jnp.bfloat16),
                           preferred_element_type=jnp.float32) + l2b_ref[...]


_K2 = 6400


def kernel(x, edge_index, edge_attr, batch, c1_w1, c1_b1, c1_g, c1_bb, c1_w2, c1_b2, p1_w, c2_w1, c2_b1, c2_g, c2_bb, c2_w2, c2_b2, p2_w, l1_w, l1_b, l2_w, l2_b):
    src, dst = edge_index[0], edge_index[1]
    f32 = jnp.float32

    xt1 = jnp.zeros((_D1, _NP1), f32).at[:3, :N_NODES].set(x.T)
    ea_t = edge_attr.T.reshape(-1)
    c1 = jnp.maximum(jnp.max(xt1, axis=1) + jnp.max(edge_attr, axis=0), 0.0) + EPS
    cb32 = jnp.tile(jnp.broadcast_to(c1[:, None], (_D1, 16)), (8, 1))

    parts = _conv1_aggr_sc(xt1.reshape(-1), ea_t, src, dst, cb32).reshape(32, 2, _NP1)

    p1n = p1_w[None, :]
    n1 = jnp.linalg.norm(p1_w).reshape(1, 1)
    h1t, sc1 = pl.pallas_call(
        _k2_body,
        out_shape=(jax.ShapeDtypeStruct((_D2, _NP1), f32),
                   jax.ShapeDtypeStruct((1, _NP1), f32)),
    )(parts, xt1, c1_w1, c1_b1[:, None], c1_g[:, None], c1_bb[:, None],
      c1_w2, c1_b2[:, None], p1n, n1)

    perm1 = jax.lax.top_k(sc1[0], _K1)[1]

    xnt, src2, dst2, st1 = _pool1_sc(h1t.reshape(-1), sc1[0], perm1, src, dst)
    st1 = st1.reshape(32, _CPT, 2, 16)
    x1max = jnp.max(st1[:, :, 0, :], axis=-1).reshape(_D2)
    x1mean = (jnp.sum(st1[:, :, 1, :], axis=-1) / _K1).reshape(_D2)
    x1c = jnp.concatenate([x1max, x1mean])[:, None]
    c2 = jnp.maximum(x1max, 0.0) + EPS

    aggr2 = _conv2_aggr_sc(xnt, src2, dst2, c2).reshape(_D2, _NP2)

    p2n = p2_w[None, :]
    n2 = jnp.linalg.norm(p2_w).reshape(1, 1)
    h2t, sc2 = pl.pallas_call(
        _k4_body,
        out_shape=(jax.ShapeDtypeStruct((_D2, _NP2), f32),
                   jax.ShapeDtypeStruct((1, _NP2), f32)),
    )(xnt.reshape(_D2, _NP2), aggr2, c2_w1, c2_b1[:, None], c2_g[:, None], c2_bb[:, None],
      c2_w2, c2_b2[:, None], p2n, n2)

    perm2 = jax.lax.top_k(sc2[0], _K2)[1]
    mask2 = jnp.zeros((1, _NP2), f32).at[0, perm2].set(1.0)

    out = pl.pallas_call(
        _k6_body,
        out_shape=jax.ShapeDtypeStruct((2, 1), f32),
    )(h2t, sc2, mask2, x1c, l1_w, l1_b[:, None], l2_w, l2_b[:, None])
    return out.reshape(1, 2)


# speedup vs baseline: 1.0653x; 1.0010x over previous
import jax
import jax.numpy as jnp
from jax import lax
from jax.experimental import pallas as pl
from jax.experimental.pallas import tpu as pltpu
from jax.experimental.pallas import tpu_sc as plsc

N_NODES = 10000
EPS = 1e-7
_NC, _NS = 2, 16
_E = 320000

_D2 = 128
_CPT = 4
_NP2 = 8016
_ECH = 8000
_NCHUNK = _E // _ECH

_D1 = 4
_NP1 = 10016
_ES = 8
_EPT = _E // _ES
_ECH1 = 8000
_NCH1 = _EPT // _ECH1
_GRP1 = _ECH1 // 16

_K1 = 8000
_EPT_B = _E // 32
_ECHB = 2000
_NCHB = _EPT_B // _ECHB
_GRPB = _ECHB // 16

_UNROLL = 8


def _fast_exp(x):
    y = jnp.maximum(x * 1.4426950408889634, -125.0)
    t = y + 12582912.0
    ni = plsc.bitcast(t, jnp.int32) - 0x4B400000
    f = y - (t - 12582912.0)
    p = 0.0013333558146428443 * f + 0.009618129107628477
    p = p * f + 0.05550410866482158
    p = p * f + 0.24022650695910072
    p = p * f + 0.6931471805599453
    p = p * f + 1.0
    scale = plsc.bitcast(lax.shift_left(ni + 127, 23), jnp.float32)
    return p * scale


def _conv2_body(xt_ref, src_ref, dst_ref, c_ref, out_ref, xv, srcv, dstv, cv, denv, numv):
    wid = lax.axis_index("s") * _NC + lax.axis_index("c")
    ch0 = wid * _CPT

    pltpu.sync_copy(xt_ref.at[pl.ds(pl.multiple_of(ch0 * _NP2, 8), _CPT * _NP2)], xv)
    pltpu.sync_copy(c_ref, cv)
    cls = [plsc.load_gather(cv, [jnp.full((16,), 0, jnp.int32) + (ch0 + q)])
           for q in range(_CPT)]

    def zbody(i, _):
        o = pl.ds(pl.multiple_of(i * 16, 8), 16)
        denv[o] = jnp.zeros((16,), jnp.float32)
        numv[o] = jnp.zeros((16,), jnp.float32)
        return 0
    lax.fori_loop(0, _CPT * _NP2 // 16, zbody, 0, unroll=_UNROLL)

    def chunk(ic, _):
        off = pl.ds(pl.multiple_of(ic * _ECH, 8), _ECH)
        pltpu.sync_copy(src_ref.at[off], srcv)
        pltpu.sync_copy(dst_ref.at[off], dstv)

        @plsc.parallel_loop(0, _ECH // 16, 1, unroll=4, carry=jnp.int32(0))
        def grp(g, j):
            sl = pl.ds(pl.multiple_of(g * 16, 8), 16)
            sv = srcv[sl]
            dv = dstv[sl]
            for q in range(_CPT):
                xg = plsc.load_gather(xv, [sv + q * _NP2])
                msg = jnp.maximum(xg, 0.0) + EPS
                w = jnp.exp(msg - cls[q])
                sidx = dv + q * _NP2
                plsc.addupdate_scatter(denv, [sidx], w)
                plsc.addupdate_scatter(numv, [sidx], w * msg)
            return j
        return 0
    lax.fori_loop(0, _NCHUNK, chunk, 0)

    def dbody(i, _):
        o = pl.ds(pl.multiple_of(i * 16, 8), 16)
        numv[o] = numv[o] / jnp.maximum(denv[o], 1e-30)
        return 0
    lax.fori_loop(0, _CPT * _NP2 // 16, dbody, 0, unroll=_UNROLL)
    pltpu.sync_copy(numv, out_ref.at[pl.ds(pl.multiple_of(ch0 * _NP2, 8), _CPT * _NP2)])


def _conv2_aggr_sc(xt_flat, src, dst, c):
    mesh = plsc.VectorSubcoreMesh(core_axis_name="c", subcore_axis_name="s",
                                  num_cores=_NC, num_subcores=_NS)
    f = pl.kernel(
        _conv2_body,
        out_type=jax.ShapeDtypeStruct((_D2 * _NP2,), jnp.float32),
        mesh=mesh,
        compiler_params=pltpu.CompilerParams(needs_layout_passes=False),
        scratch_types=[
            pltpu.VMEM((_CPT * _NP2,), jnp.float32),
            pltpu.VMEM((_ECH,), jnp.int32),
            pltpu.VMEM((_ECH,), jnp.int32),
            pltpu.VMEM((_D2,), jnp.float32),
            pltpu.VMEM((_CPT * _NP2,), jnp.float32),
            pltpu.VMEM((_CPT * _NP2,), jnp.float32),
        ],
    )
    return f(xt_flat, src, dst, c)


def _conv1_body(xt_ref, ea_ref, src_ref, dst_ref, cb_ref, out_ref,
                xv, srcv, dstv, eav, cv, denv, numv):
    wid = lax.axis_index("s") * _NC + lax.axis_index("c")
    q = wid & 3
    es = lax.shift_right_logical(wid, 2)

    pltpu.sync_copy(xt_ref.at[pl.ds(pl.multiple_of(q * _NP1, 8), _NP1)], xv)
    pltpu.sync_copy(cb_ref.at[wid], cv)
    cl = cv[...]

    def zbody(i, _):
        o = pl.ds(pl.multiple_of(i * 16, 8), 16)
        denv[o] = jnp.zeros((16,), jnp.float32)
        numv[o] = jnp.zeros((16,), jnp.float32)
        return 0
    lax.fori_loop(0, _NP1 // 16, zbody, 0, unroll=_UNROLL)

    ebase = es * _EPT

    def chunk(ic, _):
        off = ebase + ic * _ECH1
        pltpu.sync_copy(src_ref.at[pl.ds(pl.multiple_of(off, 8), _ECH1)], srcv)
        pltpu.sync_copy(dst_ref.at[pl.ds(pl.multiple_of(off, 8), _ECH1)], dstv)
        pltpu.sync_copy(ea_ref.at[pl.ds(pl.multiple_of(q * _E + off, 8), _ECH1)], eav)

        @plsc.parallel_loop(0, _GRP1, 1, unroll=8, carry=jnp.int32(0))
        def grp(g, j):
            sl = pl.ds(pl.multiple_of(g * 16, 8), 16)
            sv = srcv[sl]
            dv = dstv[sl]
            ev = eav[sl]
            xg = plsc.load_gather(xv, [sv])
            msg = jnp.maximum(xg + ev, 0.0) + EPS
            w = _fast_exp(msg - cl)
            plsc.addupdate_scatter(denv, [dv], w)
            plsc.addupdate_scatter(numv, [dv], w * msg)
            return j
        return 0
    lax.fori_loop(0, _NCH1, chunk, 0)

    obase = pl.multiple_of(wid * 2 * _NP1, 8)
    pltpu.sync_copy(denv, out_ref.at[pl.ds(obase, _NP1)])
    pltpu.sync_copy(numv, out_ref.at[pl.ds(pl.multiple_of(obase + _NP1, 8), _NP1)])


def _conv1_aggr_sc(xt_flat, ea_flat, src, dst, cb):
    mesh = plsc.VectorSubcoreMesh(core_axis_name="c", subcore_axis_name="s",
                                  num_cores=_NC, num_subcores=_NS)
    f = pl.kernel(
        _conv1_body,
        out_type=jax.ShapeDtypeStruct((32 * 2 * _NP1,), jnp.float32),
        mesh=mesh,
        compiler_params=pltpu.CompilerParams(needs_layout_passes=False),
        scratch_types=[
            pltpu.VMEM((_NP1,), jnp.float32),
            pltpu.VMEM((_ECH1,), jnp.int32),
            pltpu.VMEM((_ECH1,), jnp.int32),
            pltpu.VMEM((_ECH1,), jnp.float32),
            pltpu.VMEM((16,), jnp.float32),
            pltpu.VMEM((_NP1,), jnp.float32),
            pltpu.VMEM((_NP1,), jnp.float32),
        ],
    )
    return f(xt_flat, ea_flat, src, dst, cb)


def _pool1_body(ht_ref, sc_ref, perm_ref, src_ref, dst_ref,
                xnt_ref, s2_ref, d2_ref, st_ref,
                xv, scv, permv, nidxv, onv, srcv, dstv, s2v, d2v, statv):
    wid = lax.axis_index("s") * _NC + lax.axis_index("c")
    ch0 = wid * _CPT
    iota = lax.iota(jnp.int32, 16)

    pltpu.sync_copy(ht_ref.at[pl.ds(pl.multiple_of(ch0 * _NP1, 8), _CPT * _NP1)], xv)
    pltpu.sync_copy(sc_ref, scv)
    pltpu.sync_copy(perm_ref, permv)

    def zb(i, _):
        nidxv[pl.ds(pl.multiple_of(i * 16, 8), 16)] = jnp.full((16,), _K1, jnp.int32)
        return 0
    lax.fori_loop(0, _NP1 // 16, zb, 0, unroll=_UNROLL)

    def sb(g, _):
        pv = permv[pl.ds(pl.multiple_of(g * 16, 8), 16)]
        plsc.store_scatter(nidxv, [pv], g * 16 + iota)
        return 0
    lax.fori_loop(0, _K1 // 16, sb, 0, unroll=_UNROLL)

    for q in range(_CPT):
        def gb(g, carry):
            mx, sm = carry
            pv = permv[pl.ds(pl.multiple_of(g * 16, 8), 16)]
            s = plsc.load_gather(scv, [pv])
            xg = plsc.load_gather(xv, [q * _NP1 + pv])
            v = xg * s
            onv[pl.ds(pl.multiple_of(q * _NP2 + g * 16, 8), 16)] = v
            return (jnp.maximum(mx, v), sm + v)
        mx, sm = lax.fori_loop(0, _K1 // 16, gb,
                               (jnp.full((16,), -3.4e38, jnp.float32),
                                jnp.zeros((16,), jnp.float32)), unroll=_UNROLL)
        onv[pl.ds(q * _NP2 + _K1, 16)] = jnp.zeros((16,), jnp.float32)
        statv[pl.ds(q * 32, 16)] = mx
        statv[pl.ds(q * 32 + 16, 16)] = sm

    pltpu.sync_copy(onv, xnt_ref.at[pl.ds(pl.multiple_of(ch0 * _NP2, 8), _CPT * _NP2)])
    pltpu.sync_copy(statv, st_ref.at[wid])

    ebase = wid * _EPT_B

    def chunk(ic, _):
        off = ebase + ic * _ECHB
        osl = pl.ds(pl.multiple_of(off, 8), _ECHB)
        pltpu.sync_copy(src_ref.at[osl], srcv)
        pltpu.sync_copy(dst_ref.at[osl], dstv)

        def grp(g, _):
            sl = pl.ds(pl.multiple_of(g * 16, 8), 16)
            ns = plsc.load_gather(nidxv, [srcv[sl]])
            nd = plsc.load_gather(nidxv, [dstv[sl]])
            bad = (ns == _K1) | (nd == _K1)
            s2v[sl] = jnp.where(bad, _K1, ns)
            d2v[sl] = jnp.where(bad, _K1, nd)
            return 0
        lax.fori_loop(0, _GRPB, grp, 0, unroll=_UNROLL)
        pltpu.sync_copy(s2v, s2_ref.at[osl])
        pltpu.sync_copy(d2v, d2_ref.at[osl])
        return 0
    lax.fori_loop(0, _NCHB, chunk, 0)


def _pool1_sc(ht_flat, score, perm, src, dst):
    mesh = plsc.VectorSubcoreMesh(core_axis_name="c", subcore_axis_name="s",
                                  num_cores=_NC, num_subcores=_NS)
    f = pl.kernel(
        _pool1_body,
        out_type=(jax.ShapeDtypeStruct((_D2 * _NP2,), jnp.float32),
                  jax.ShapeDtypeStruct((_E,), jnp.int32),
                  jax.ShapeDtypeStruct((_E,), jnp.int32),
                  jax.ShapeDtypeStruct((32, 128), jnp.float32)),
        mesh=mesh,
        compiler_params=pltpu.CompilerParams(needs_layout_passes=False),
        scratch_types=[
            pltpu.VMEM((_CPT * _NP1,), jnp.float32),
            pltpu.VMEM((_NP1,), jnp.float32),
            pltpu.VMEM((_K1,), jnp.int32),
            pltpu.VMEM((_NP1,), jnp.int32),
            pltpu.VMEM((_CPT * _NP2,), jnp.float32),
            pltpu.VMEM((_ECHB,), jnp.int32),
            pltpu.VMEM((_ECHB,), jnp.int32),
            pltpu.VMEM((_ECHB,), jnp.int32),
            pltpu.VMEM((_ECHB,), jnp.int32),
            pltpu.VMEM((128,), jnp.float32),
        ],
    )
    return f(ht_flat, score, perm, src, dst)


def _k2_body(parts_ref, xt_ref, w1_ref, b1_ref, g_ref, bb_ref, w2_ref, b2_ref, pn_ref, nrm_ref,
             ht_ref, sc_ref):
    ps = parts_ref[...]
    dn = jnp.sum(ps.reshape(_ES, _D1, 2, _NP1), axis=0)
    aggr = dn[:, 1, :] / jnp.maximum(dn[:, 0, :], 1e-30)
    out1 = xt_ref[...] + aggr
    h = jnp.dot(w1_ref[...].astype(jnp.bfloat16), out1.astype(jnp.bfloat16),
                preferred_element_type=jnp.float32) + b1_ref[...]
    mask = lax.broadcasted_iota(jnp.int32, (1, _NP1), 1) < N_NODES
    hm = jnp.where(mask, h, 0.0)
    mu = jnp.sum(hm, axis=1, keepdims=True) / N_NODES
    d = jnp.where(mask, h - mu, 0.0)
    var = jnp.sum(d * d, axis=1, keepdims=True) / N_NODES
    v = var + 1e-5
    r = lax.rsqrt(v)
    r = r * (1.5 - 0.5 * v * r * r)
    hn = (h - mu) * r * g_ref[...] + bb_ref[...]
    hr = jnp.maximum(hn, 0.0)
    h2 = jnp.maximum(jnp.dot(w2_ref[...].astype(jnp.bfloat16), hr.astype(jnp.bfloat16),
                             preferred_element_type=jnp.float32) + b2_ref[...], 0.0)
    ht_ref[...] = h2
    s = jnp.dot(pn_ref[...].astype(jnp.bfloat16), h2.astype(jnp.bfloat16),
                preferred_element_type=jnp.float32) / nrm_ref[...]
    sc_ref[...] = jnp.where(mask, jnp.tanh(s), -1e30)


def _k4_body(xnt_ref, aggr_ref, w1_ref, b1_ref, g_ref, bb_ref, w2_ref, b2_ref, pn_ref, nrm_ref,
             ht_ref, sc_ref):
    out2 = xnt_ref[...] + aggr_ref[...]
    h = jnp.dot(w1_ref[...].astype(jnp.bfloat16), out2.astype(jnp.bfloat16),
                preferred_element_type=jnp.float32) + b1_ref[...]
    mask = lax.broadcasted_iota(jnp.int32, (1, _NP2), 1) < _K1
    hm = jnp.where(mask, h, 0.0)
    mu = jnp.sum(hm, axis=1, keepdims=True) / _K1
    d = jnp.where(mask, h - mu, 0.0)
    var = jnp.sum(d * d, axis=1, keepdims=True) / _K1
    v = var + 1e-5
    r = lax.rsqrt(v)
    r = r * (1.5 - 0.5 * v * r * r)
    hn = (h - mu) * r * g_ref[...] + bb_ref[...]
    hr = jnp.maximum(hn, 0.0)
    h2 = jnp.maximum(jnp.dot(w2_ref[...].astype(jnp.bfloat16), hr.astype(jnp.bfloat16),
                             preferred_element_type=jnp.float32) + b2_ref[...], 0.0)
    ht_ref[...] = h2
    s = jnp.dot(pn_ref[...].astype(jnp.bfloat16), h2.astype(jnp.bfloat16),
                preferred_element_type=jnp.float32) / nrm_ref[...]
    sc_ref[...] = jnp.where(mask, jnp.tanh(s), -1e30)


def _k6_body(ht_ref, sc_ref, mask_ref, x1_ref, l1w_ref, l1b_ref, l2w_ref, l2b_ref, out_ref):
    v = ht_ref[...] * sc_ref[...]
    m = mask_ref[...]
    x2max = jnp.max(jnp.where(m > 0, v, -3.4e38), axis=1, keepdims=True)
    x2mean = jnp.sum(v * m, axis=1, keepdims=True) / _K2
    z = x1_ref[...] + jnp.concatenate([x2max, x2mean], axis=0)
    h = jnp.maximum(jnp.dot(l1w_ref[...].astype(jnp.bfloat16), z.astype(jnp.bfloat16),
                            preferred_element_type=jnp.float32) + l1b_ref[...], 0.0)
    out_ref[...] = jnp.dot(l2w_ref[...].astype(jnp.bfloat16), h.astype(jnp.bfloat16),
                           preferred_element_type=jnp.float32) + l2b_ref[...]


_K2 = 6400


def kernel(x, edge_index, edge_attr, batch, c1_w1, c1_b1, c1_g, c1_bb, c1_w2, c1_b2, p1_w, c2_w1, c2_b1, c2_g, c2_bb, c2_w2, c2_b2, p2_w, l1_w, l1_b, l2_w, l2_b):
    src, dst = edge_index[0], edge_index[1]
    f32 = jnp.float32

    xt1 = jnp.zeros((_D1, _NP1), f32).at[:3, :N_NODES].set(x.T)
    ea_t = edge_attr.T.reshape(-1)
    c1 = jnp.maximum(jnp.max(xt1, axis=1) + jnp.max(edge_attr, axis=0), 0.0) + EPS
    cb32 = jnp.tile(jnp.broadcast_to(c1[:, None], (_D1, 16)), (8, 1))

    parts = _conv1_aggr_sc(xt1.reshape(-1), ea_t, src, dst, cb32).reshape(32, 2, _NP1)

    p1n = p1_w[None, :]
    n1 = jnp.linalg.norm(p1_w).reshape(1, 1)
    h1t, sc1 = pl.pallas_call(
        _k2_body,
        out_shape=(jax.ShapeDtypeStruct((_D2, _NP1), f32),
                   jax.ShapeDtypeStruct((1, _NP1), f32)),
    )(parts, xt1, c1_w1, c1_b1[:, None], c1_g[:, None], c1_bb[:, None],
      c1_w2, c1_b2[:, None], p1n, n1)

    perm1 = jax.lax.top_k(sc1[0], _K1)[1]

    xnt, src2, dst2, st1 = _pool1_sc(h1t.reshape(-1), sc1[0], perm1, src, dst)
    st1 = st1.reshape(32, _CPT, 2, 16)
    x1max = jnp.max(st1[:, :, 0, :], axis=-1).reshape(_D2)
    x1mean = (jnp.sum(st1[:, :, 1, :], axis=-1) / _K1).reshape(_D2)
    x1c = jnp.concatenate([x1max, x1mean])[:, None]
    c2 = jnp.maximum(x1max, 0.0) + EPS

    aggr2 = _conv2_aggr_sc(xnt, src2, dst2, c2).reshape(_D2, _NP2)

    p2n = p2_w[None, :]
    n2 = jnp.linalg.norm(p2_w).reshape(1, 1)
    h2t, sc2 = pl.pallas_call(
        _k4_body,
        out_shape=(jax.ShapeDtypeStruct((_D2, _NP2), f32),
                   jax.ShapeDtypeStruct((1, _NP2), f32)),
    )(xnt.reshape(_D2, _NP2), aggr2, c2_w1, c2_b1[:, None], c2_g[:, None], c2_bb[:, None],
      c2_w2, c2_b2[:, None], p2n, n2)

    perm2 = jax.lax.top_k(sc2[0], _K2)[1]
    mask2 = jnp.zeros((1, _NP2), f32).at[0, perm2].set(1.0)

    out = pl.pallas_call(
        _k6_body,
        out_shape=jax.ShapeDtypeStruct((2, 1), f32),
    )(h2t, sc2, mask2, x1c, l1_w, l1_b[:, None], l2_w, l2_b[:, None])
    return out.reshape(1, 2)
